# baseline (speedup 1.0000x reference)
import functools

import jax
import jax.numpy as jnp
from jax import lax
from jax.experimental import pallas as pl
from jax.experimental.pallas import tpu as pltpu
from jax.experimental.pallas import tpu_sc as plsc

B = 4096 * 50
D = 64
NC, NS = 2, 16
NW = NC * NS
BPW = B // NW
CHUNK = 800
NCHUNK = BPW // CHUNK


@functools.lru_cache(maxsize=None)
def _build():
    mesh = plsc.VectorSubcoreMesh(core_axis_name="c", subcore_axis_name="s")

    @functools.partial(
        pl.kernel,
        out_type=jax.ShapeDtypeStruct((B, D), jnp.float32),
        mesh=mesh,
        compiler_params=pltpu.CompilerParams(use_tc_tiling_on_sc=False),
        scratch_types=[
            pltpu.VMEM((BPW,), jnp.int32),
            pltpu.VMEM((CHUNK, D), jnp.float32),
            pltpu.VMEM((CHUNK, D), jnp.float32),
            pltpu.SemaphoreType.DMA,
            pltpu.SemaphoreType.DMA,
        ],
    )
    def gather_kernel(idx_hbm, table_hbm, out_hbm, idx_v, rows0, rows1,
                      sem0, sem1):
        wid = lax.axis_index("s") * NC + lax.axis_index("c")
        base = wid * BPW
        pltpu.sync_copy(idx_hbm.at[pl.ds(base, BPW)], idx_v)

        rows = (rows0, rows1)
        sems = (sem0, sem1)
        pending = [None, None]
        pending[0] = pltpu.async_copy(
            table_hbm.at[idx_v.at[pl.ds(0, CHUNK)]], rows0, sem0)
        for g in range(NCHUNK):
            b = g % 2
            pending[b].wait()
            if g + 1 < NCHUNK:
                nb = 1 - b
                pending[nb] = pltpu.async_copy(
                    table_hbm.at[idx_v.at[pl.ds((g + 1) * CHUNK, CHUNK)]],
                    rows[nb], sems[nb])
            pltpu.sync_copy(rows[b], out_hbm.at[pl.ds(base + g * CHUNK, CHUNK)])

    return gather_kernel


def kernel(src_seq, emb_weight):
    idx = src_seq.reshape(-1).astype(jnp.int32)
    out = _build()(idx, emb_weight)
    return out.reshape(src_seq.shape + (emb_weight.shape[-1],))
